# baseline (speedup 1.0000x reference)
import jax
import jax.numpy as jnp
from jax import lax
from jax.experimental import pallas as pl
from jax.experimental.pallas import tpu as pltpu
from jax.experimental.pallas import tpu_sc as plsc

N = 100000
D = 128
NC = 2
NS = 16
NW = NC * NS
CHUNK = 3136
BETA = 1.0


def _body(x_hbm, p_hbm, im_hbm, part_hbm, psrc_v, p_v, im_v, out_v, shared, all_v, sem, semb, semc):
    c = lax.axis_index("c")
    s = lax.axis_index("s")
    wid = s * NC + c
    start = wid * CHUNK
    base = jnp.minimum(start, N - CHUNK)
    half = CHUNK // 2

    cpg0 = pltpu.async_copy(
        x_hbm.at[pl.ds(base, half), pl.ds(D - 16, 16)],
        psrc_v.at[pl.ds(0, half)],
        semb,
    )
    cpg1 = pltpu.async_copy(
        x_hbm.at[pl.ds(base + half, half), pl.ds(D - 16, 16)],
        psrc_v.at[pl.ds(half, half)],
        semc,
    )
    cp1 = pltpu.async_copy(p_hbm.at[pl.ds(base, CHUNK)], p_v, sem)
    cp2 = pltpu.async_copy(im_hbm.at[pl.ds(base, CHUNK)], im_v, sem)

    cpg0.wait()
    cp1.wait()
    cp2.wait()

    iota = lax.iota(jnp.int32, 16)
    lane15 = jnp.full((16,), 15, jnp.int32)
    lane0 = jnp.zeros((16,), jnp.int32)
    zero_f = jnp.zeros((16,), jnp.float32)
    one_f = jnp.ones((16,), jnp.float32)

    def step(off, carry):
        a_dp, a_np, a_iz, a_nz = carry
        idx = off + iota
        ps = plsc.load_gather(psrc_v, [idx, lane15])
        p = plsc.load_gather(p_v, [idx, lane0])
        im = im_v[pl.ds(off, 16)]
        valid = (base + idx) >= start
        mpos = jnp.where(jnp.logical_and(ps > 0, valid), one_f, zero_f)
        mzero = jnp.where(jnp.logical_and(ps == 0, valid), one_f, zero_f)
        d = ps - p
        return (
            a_dp + d * d * mpos,
            a_np + mpos,
            a_iz + im * im * mzero,
            a_nz + mzero,
        )

    acc = (zero_f, zero_f, zero_f, zero_f)
    acc = plsc.parallel_loop(0, half, 16, unroll=4, carry=acc)(step)
    cpg1.wait()
    a_dp, a_np, a_iz, a_nz = plsc.parallel_loop(
        half, CHUNK, 16, unroll=4, carry=acc
    )(step)

    s_dp = jnp.sum(a_dp)
    s_np = jnp.sum(a_np)
    s_iz = jnp.sum(a_iz)
    s_nz = jnp.sum(a_nz)

    res = (
        jnp.where(iota == 0, s_dp, 0.0)
        + jnp.where(iota == 1, s_np, 0.0)
        + jnp.where(iota == 2, s_iz, 0.0)
        + jnp.where(iota == 3, s_nz, 0.0)
    )
    out_v[...] = res
    pltpu.sync_copy(out_v, shared.at[s])
    plsc.subcore_barrier()

    @pl.when(s == 0)
    def _():
        pltpu.sync_copy(shared, all_v)
        tot = all_v[0]
        for r in range(1, NS):
            tot = tot + all_v[r]
        out_v[...] = tot
        pltpu.sync_copy(out_v, part_hbm.at[c])


@jax.jit
def _partials(x, P, imbalance):
    mesh = plsc.VectorSubcoreMesh(
        core_axis_name="c", subcore_axis_name="s", num_cores=NC, num_subcores=NS
    )
    return pl.kernel(
        _body,
        out_type=jax.ShapeDtypeStruct((NC, 16), jnp.float32),
        mesh=mesh,
        scratch_types=[
            pltpu.VMEM((CHUNK, 16), jnp.float32),
            pltpu.VMEM((CHUNK, 1), jnp.float32),
            pltpu.VMEM((CHUNK,), jnp.float32),
            pltpu.VMEM((16,), jnp.float32),
            pltpu.VMEM_SHARED((NS, 16), jnp.float32),
            pltpu.VMEM((NS, 16), jnp.float32),
            pltpu.SemaphoreType.DMA,
            pltpu.SemaphoreType.DMA,
            pltpu.SemaphoreType.DMA,
        ],
        compiler_params=pltpu.CompilerParams(
            use_tc_tiling_on_sc=False, needs_layout_passes=False
        ),
    )(x, P, imbalance)


def kernel(x, P, imbalance):
    part = _partials(x, P, imbalance)
    sums = part[0] + part[1]
    pl_ = sums[0] / sums[1]
    ql = sums[2] / sums[3]
    return BETA * ql + (1.0 - BETA) * pl_


# speedup vs baseline: 4.1450x; 4.1450x over previous
import jax
import jax.numpy as jnp
from jax import lax
from jax.experimental import pallas as pl
from jax.experimental.pallas import tpu as pltpu
from jax.experimental.pallas import tpu_sc as plsc

N = 100000
D = 128
NC = 2
NS = 16
NW = NC * NS
CHUNK = 3136
BETA = 1.0


def _body(xf_hbm, p_hbm, im_hbm, part_hbm, idx_v, psrc_v, p_v, im_v, out_v, shared, all_v, sem, semb, semc):
    c = lax.axis_index("c")
    s = lax.axis_index("s")
    wid = s * NC + c
    start = wid * CHUNK
    base = jnp.minimum(start, N - CHUNK)
    half = CHUNK // 2

    cp1 = pltpu.async_copy(p_hbm.at[pl.ds(base, CHUNK)], p_v, sem)
    cp2 = pltpu.async_copy(im_hbm.at[pl.ds(base, CHUNK)], im_v, sem)

    iota = lax.iota(jnp.int32, 16)

    def _fill(off):
        idx_v[pl.ds(off, 16)] = (base + off + iota) * D + (D - 1)

    plsc.parallel_loop(0, half, 16, unroll=4)(_fill)
    cpg0 = pltpu.async_copy(
        xf_hbm.at[idx_v.at[pl.ds(0, half)]], psrc_v.at[pl.ds(0, half)], semb
    )
    plsc.parallel_loop(half, CHUNK, 16, unroll=4)(_fill)
    cpg1 = pltpu.async_copy(
        xf_hbm.at[idx_v.at[pl.ds(half, half)]], psrc_v.at[pl.ds(half, half)], semc
    )

    cpg0.wait()
    cp1.wait()
    cp2.wait()

    zero_f = jnp.zeros((16,), jnp.float32)
    one_f = jnp.ones((16,), jnp.float32)

    def step(off, carry):
        a_dp, a_np, a_iz, a_nz = carry
        ps = psrc_v[pl.ds(off, 16)]
        p = p_v[pl.ds(off, 16)]
        im = im_v[pl.ds(off, 16)]
        valid = (base + off + iota) >= start
        mpos = jnp.where(jnp.logical_and(ps > 0, valid), one_f, zero_f)
        mzero = jnp.where(jnp.logical_and(ps == 0, valid), one_f, zero_f)
        d = ps - p
        return (
            a_dp + d * d * mpos,
            a_np + mpos,
            a_iz + im * im * mzero,
            a_nz + mzero,
        )

    acc = (zero_f, zero_f, zero_f, zero_f)
    acc = plsc.parallel_loop(0, half, 16, unroll=4, carry=acc)(step)
    cpg1.wait()
    a_dp, a_np, a_iz, a_nz = plsc.parallel_loop(
        half, CHUNK, 16, unroll=4, carry=acc
    )(step)

    s_dp = jnp.sum(a_dp)
    s_np = jnp.sum(a_np)
    s_iz = jnp.sum(a_iz)
    s_nz = jnp.sum(a_nz)

    res = (
        jnp.where(iota == 0, s_dp, 0.0)
        + jnp.where(iota == 1, s_np, 0.0)
        + jnp.where(iota == 2, s_iz, 0.0)
        + jnp.where(iota == 3, s_nz, 0.0)
    )
    out_v[...] = res
    pltpu.sync_copy(out_v, shared.at[s])
    plsc.subcore_barrier()

    @pl.when(s == 0)
    def _():
        pltpu.sync_copy(shared, all_v)
        tot = all_v[0]
        for r in range(1, NS):
            tot = tot + all_v[r]
        out_v[...] = tot
        pltpu.sync_copy(out_v, part_hbm.at[c])


@jax.jit
def _partials(x_flat, p_flat, imbalance):
    mesh = plsc.VectorSubcoreMesh(
        core_axis_name="c", subcore_axis_name="s", num_cores=NC, num_subcores=NS
    )
    return pl.kernel(
        _body,
        out_type=jax.ShapeDtypeStruct((NC, 16), jnp.float32),
        mesh=mesh,
        scratch_types=[
            pltpu.VMEM((CHUNK,), jnp.int32),
            pltpu.VMEM((CHUNK,), jnp.float32),
            pltpu.VMEM((CHUNK,), jnp.float32),
            pltpu.VMEM((CHUNK,), jnp.float32),
            pltpu.VMEM((16,), jnp.float32),
            pltpu.VMEM_SHARED((NS, 16), jnp.float32),
            pltpu.VMEM((NS, 16), jnp.float32),
            pltpu.SemaphoreType.DMA,
            pltpu.SemaphoreType.DMA,
            pltpu.SemaphoreType.DMA,
        ],
        compiler_params=pltpu.CompilerParams(
            use_tc_tiling_on_sc=False, needs_layout_passes=False
        ),
    )(x_flat, p_flat, imbalance)


def kernel(x, P, imbalance):
    part = _partials(x.reshape(-1), P.reshape(-1), imbalance)
    sums = part[0] + part[1]
    pl_ = sums[0] / sums[1]
    ql = sums[2] / sums[3]
    return BETA * ql + (1.0 - BETA) * pl_


# speedup vs baseline: 4.1854x; 1.0097x over previous
import jax
import jax.numpy as jnp
from jax import lax
from jax.experimental import pallas as pl
from jax.experimental.pallas import tpu as pltpu
from jax.experimental.pallas import tpu_sc as plsc

N = 100000
D = 128
NC = 2
NS = 16
NW = NC * NS
CHUNK = 3136
BETA = 1.0


def _body(xf_hbm, p_hbm, im_hbm, part_hbm, idx_v, psrc_v, p_v, im_v, out_v, shared, all_v, sem, semb, semc):
    c = lax.axis_index("c")
    s = lax.axis_index("s")
    wid = s * NC + c
    start = wid * CHUNK
    base = jnp.minimum(start, N - CHUNK)
    half = CHUNK // 2

    cp1 = pltpu.async_copy(p_hbm.at[pl.ds(base, CHUNK)], p_v, sem)
    cp2 = pltpu.async_copy(im_hbm.at[pl.ds(base, CHUNK)], im_v, sem)

    iota = lax.iota(jnp.int32, 16)

    def _fill(off):
        idx_v[pl.ds(off, 16)] = (base + off + iota) * D + (D - 1)

    plsc.parallel_loop(0, half, 16, unroll=2)(_fill)
    cpg0 = pltpu.async_copy(
        xf_hbm.at[idx_v.at[pl.ds(0, half)]], psrc_v.at[pl.ds(0, half)], semb
    )
    plsc.parallel_loop(half, CHUNK, 16, unroll=2)(_fill)
    cpg1 = pltpu.async_copy(
        xf_hbm.at[idx_v.at[pl.ds(half, half)]], psrc_v.at[pl.ds(half, half)], semc
    )

    cpg0.wait()
    cp1.wait()
    cp2.wait()

    zero_f = jnp.zeros((16,), jnp.float32)
    one_f = jnp.ones((16,), jnp.float32)

    def step(off, carry):
        a_dp, a_np, a_iz, a_nz = carry
        ps = psrc_v[pl.ds(off, 16)]
        p = p_v[pl.ds(off, 16)]
        im = im_v[pl.ds(off, 16)]
        valid = (base + off + iota) >= start
        mpos = jnp.where(jnp.logical_and(ps > 0, valid), one_f, zero_f)
        mzero = jnp.where(jnp.logical_and(ps == 0, valid), one_f, zero_f)
        d = ps - p
        return (
            a_dp + d * d * mpos,
            a_np + mpos,
            a_iz + im * im * mzero,
            a_nz + mzero,
        )

    acc = (zero_f, zero_f, zero_f, zero_f)
    acc = plsc.parallel_loop(0, half, 16, unroll=2, carry=acc)(step)
    cpg1.wait()
    a_dp, a_np, a_iz, a_nz = plsc.parallel_loop(
        half, CHUNK, 16, unroll=2, carry=acc
    )(step)

    s_dp = jnp.sum(a_dp)
    s_np = jnp.sum(a_np)
    s_iz = jnp.sum(a_iz)
    s_nz = jnp.sum(a_nz)

    res = (
        jnp.where(iota == 0, s_dp, 0.0)
        + jnp.where(iota == 1, s_np, 0.0)
        + jnp.where(iota == 2, s_iz, 0.0)
        + jnp.where(iota == 3, s_nz, 0.0)
    )
    out_v[...] = res
    pltpu.sync_copy(out_v, shared.at[s])
    plsc.subcore_barrier()

    @pl.when(s == 0)
    def _():
        pltpu.sync_copy(shared, all_v)
        tot = all_v[0]
        for r in range(1, NS):
            tot = tot + all_v[r]
        out_v[...] = tot
        pltpu.sync_copy(out_v, part_hbm.at[c])


@jax.jit
def _partials(x_flat, p_flat, imbalance):
    mesh = plsc.VectorSubcoreMesh(
        core_axis_name="c", subcore_axis_name="s", num_cores=NC, num_subcores=NS
    )
    return pl.kernel(
        _body,
        out_type=jax.ShapeDtypeStruct((NC, 16), jnp.float32),
        mesh=mesh,
        scratch_types=[
            pltpu.VMEM((CHUNK,), jnp.int32),
            pltpu.VMEM((CHUNK,), jnp.float32),
            pltpu.VMEM((CHUNK,), jnp.float32),
            pltpu.VMEM((CHUNK,), jnp.float32),
            pltpu.VMEM((16,), jnp.float32),
            pltpu.VMEM_SHARED((NS, 16), jnp.float32),
            pltpu.VMEM((NS, 16), jnp.float32),
            pltpu.SemaphoreType.DMA,
            pltpu.SemaphoreType.DMA,
            pltpu.SemaphoreType.DMA,
        ],
        compiler_params=pltpu.CompilerParams(
            use_tc_tiling_on_sc=False, needs_layout_passes=False
        ),
    )(x_flat, p_flat, imbalance)


def kernel(x, P, imbalance):
    part = _partials(x.reshape(-1), P.reshape(-1), imbalance)
    sums = part[0] + part[1]
    pl_ = sums[0] / sums[1]
    ql = sums[2] / sums[3]
    return BETA * ql + (1.0 - BETA) * pl_
